# baseline (speedup 1.0000x reference)
import jax
import jax.numpy as jnp
from jax import lax
from jax.experimental import pallas as pl
from jax.experimental.pallas import tpu as pltpu

_N_GROUPS = 1000
_LEAVES = 100
_BATCH = 1024
_PRED_DIM = _N_GROUPS + _N_GROUPS * _LEAVES
_ROOT_LS = 0.1
_GROUP_LS = 0.05

_W = 128
_R = 16
_NB = _BATCH // _R
_PADC = 1024


def _body(c0_ref, xs_ref, *rest):
  win_refs = rest[:2 * _R]
  gwp_ref, g_ref, y_ref, rw_ref, al_ref, out_ref = rest[2 * _R:]
  i = pl.program_id(0)
  xs = xs_ref[...]
  gcol = g_ref[0]
  ycol = y_ref[0]
  rw = rw_ref[...]
  al = al_ref[...]

  cols = lax.broadcasted_iota(jnp.int32, xs.shape, 1)
  x = jnp.where(cols < _N_GROUPS, xs, -1e30)
  m = jnp.max(x, axis=1, keepdims=True)
  lse = m + jnp.log(jnp.sum(jnp.exp(x - m), axis=1, keepdims=True))
  oh = cols == gcol
  ohf = oh.astype(jnp.float32)
  x_y = jnp.sum(jnp.where(oh, x, 0.0), axis=1, keepdims=True)
  w_y = jnp.sum(jnp.where(oh, rw, 0.0), axis=1, keepdims=True)
  a_y = jnp.sum(jnp.where(oh, al, 0.0), axis=1, keepdims=True)
  logp_y = x_y - lse
  smooth_sum = (jnp.sum(xs * rw, axis=1, keepdims=True) - lse * jnp.sum(rw))
  root_loss = (-(1.0 - _ROOT_LS) * logp_y
               - (_ROOT_LS / _N_GROUPS) * smooth_sum / w_y)

  rows = []
  for k in range(_R):
    lo = win_refs[2 * k][k % 8:k % 8 + 1, :]
    hi = win_refs[2 * k + 1][k % 8:k % 8 + 1, :]
    rows.append(jnp.concatenate([lo, hi], axis=1))
  buf = jnp.concatenate(rows, axis=0)

  p = (_N_GROUPS + gcol * _LEAVES) & (_W - 1)
  for s in (64, 32, 16, 8, 4):
    rolled = jnp.concatenate([buf[:, s:], buf[:, :s]], axis=1)
    buf = jnp.where((p & s) != 0, rolled, buf)
  glr = buf[:, :_W]

  wg = jax.lax.dot_general(ohf, gwp_ref[...], (((1,), (0,)), ((), ())),
                           preferred_element_type=jnp.float32)

  cols2 = lax.broadcasted_iota(jnp.int32, glr.shape, 1)
  gl = jnp.where(cols2 < _LEAVES, glr, -1e30)
  m2 = jnp.max(gl, axis=1, keepdims=True)
  lse2 = m2 + jnp.log(jnp.sum(jnp.exp(gl - m2), axis=1, keepdims=True))
  oh2 = cols2 == ycol
  gl_y = jnp.sum(jnp.where(oh2, gl, 0.0), axis=1, keepdims=True)
  wg_y = jnp.sum(jnp.where(oh2, wg, 0.0), axis=1, keepdims=True)
  logp_y2 = gl_y - lse2
  smooth_sum2 = (jnp.sum(gl * wg, axis=1, keepdims=True)
                 - lse2 * jnp.sum(wg, axis=1, keepdims=True))
  grp_loss = a_y * (-(1.0 - _GROUP_LS) * logp_y2
                    - (_GROUP_LS / _LEAVES) * smooth_sum2 / wg_y)

  acc = (jnp.sum(root_loss + grp_loss) * (1.0 / _BATCH)).reshape(1, 1)

  @pl.when(i == 0)
  def _():
    out_ref[...] = jnp.zeros_like(out_ref)

  out_ref[...] += acc


def _win_spec(k, hi):
  def idx(j, c0):
    return (2 * j + k // 8, c0[_R * j + k] + hi)
  return pl.BlockSpec((8, _W), idx)


def _run(c0, xs, bp, gwp, g3, y3, rw2, al2):
  win_specs = []
  for k in range(_R):
    win_specs.append(_win_spec(k, 0))
    win_specs.append(_win_spec(k, 1))
  grid_spec = pltpu.PrefetchScalarGridSpec(
      num_scalar_prefetch=1,
      grid=(_NB,),
      in_specs=[
          pl.BlockSpec((_R, _PADC), lambda j, c0: (j, 0)),
          *win_specs,
          pl.BlockSpec((_PADC, _W), lambda j, c0: (0, 0)),
          pl.BlockSpec((1, _R, 1), lambda j, c0: (j, 0, 0)),
          pl.BlockSpec((1, _R, 1), lambda j, c0: (j, 0, 0)),
          pl.BlockSpec((1, _PADC), lambda j, c0: (0, 0)),
          pl.BlockSpec((1, _PADC), lambda j, c0: (0, 0)),
      ],
      out_specs=pl.BlockSpec((1, 1), lambda j, c0: (0, 0)),
  )
  out = pl.pallas_call(
      _body,
      grid_spec=grid_spec,
      out_shape=jax.ShapeDtypeStruct((1, 1), jnp.float32),
  )(c0, xs, *([bp] * (2 * _R)), gwp, g3, y3, rw2, al2)
  return out[0, 0]


def kernel(batch_predictions, targets, root_weight, group_weights, group_alphas):
  g = (targets // _LEAVES).astype(jnp.int32)
  y = (targets % _LEAVES).astype(jnp.int32)
  c0 = (_N_GROUPS + g * _LEAVES) // _W
  xs = lax.slice(batch_predictions, (0, 0), (_BATCH, _PADC))
  gwp = jnp.pad(group_weights, ((0, _PADC - _N_GROUPS), (0, _W - _LEAVES)))
  g3 = g.reshape(_NB, _R, 1)
  y3 = y.reshape(_NB, _R, 1)
  rw2 = jnp.pad(root_weight, (0, _PADC - _N_GROUPS)).reshape(1, _PADC)
  al2 = jnp.pad(group_alphas, (0, _PADC - _N_GROUPS)).reshape(1, _PADC)
  return _run(c0, xs, batch_predictions, gwp, g3, y3, rw2, al2)


# speedup vs baseline: 1.1587x; 1.1587x over previous
import functools

import jax
import jax.numpy as jnp
from jax import lax
from jax.experimental import pallas as pl
from jax.experimental.pallas import tpu as pltpu
from jax.experimental.pallas import tpu_sc as plsc

_N_GROUPS = 1000
_LEAVES = 100
_BATCH = 1024
_PRED_DIM = _N_GROUPS + _N_GROUPS * _LEAVES
_ROOT_LS = 0.1
_GROUP_LS = 0.05

_W = 128
_WIN = 2 * _W

_NC = 2
_NS = 16
_NW = _NC * _NS
_BPW = _BATCH // _NW

_R = 128
_NB = _BATCH // _R
_PADC = 1024


def _sc_gather(bp, g, gw_pad):
  mesh = plsc.VectorSubcoreMesh(core_axis_name="c", subcore_axis_name="s")

  @functools.partial(
      pl.kernel,
      mesh=mesh,
      out_type=[
          jax.ShapeDtypeStruct((_BATCH, _WIN), jnp.float32),
          jax.ShapeDtypeStruct((_BATCH, _W), jnp.float32),
      ],
      scratch_types=[
          pltpu.VMEM((_BATCH,), jnp.int32),
          pltpu.VMEM((_BPW,), jnp.int32),
          pltpu.VMEM((_BPW, 8, _WIN), jnp.float32),
          pltpu.VMEM((_BPW, _WIN), jnp.float32),
          pltpu.VMEM((_BPW, _W), jnp.float32),
          pltpu.SemaphoreType.DMA,
          pltpu.SemaphoreType.DMA,
      ],
  )
  def k(bp_hbm, g_hbm, gw_hbm, win_out, wg_out, gall, gv, w8, wv, wgv, s1, s2):
    wid = lax.axis_index("s") * _NC + lax.axis_index("c")
    base = wid * _BPW
    pltpu.sync_copy(g_hbm, gall)
    for j in range(_BPW // 16):
      gv[pl.ds(j * 16, 16)] = gall[pl.ds(base + j * 16, 16)]
    cpw = pltpu.async_copy(gw_hbm.at[gv], wgv, s2)
    copies = []
    for j in range(_BPW // 16):
      g16 = gall[pl.ds(base + j * 16, 16)]
      c16 = lax.shift_right_logical(_N_GROUPS + g16 * _LEAVES, 7) * _W
      for t in range(16):
        s = j * 16 + t
        col = pl.multiple_of(c16[t], _W)
        copies.append(pltpu.async_copy(
            bp_hbm.at[pl.ds(base + (s // 8) * 8, 8), pl.ds(col, _WIN)],
            w8.at[s], s1))
    for cp in copies:
      cp.wait()
    cpw.wait()
    for s in range(_BPW):
      for c in range(_WIN // 16):
        wv[s, pl.ds(c * 16, 16)] = w8[s, s % 8, pl.ds(c * 16, 16)]
    pltpu.sync_copy(wv, win_out.at[pl.ds(base, _BPW)])
    pltpu.sync_copy(wgv, wg_out.at[pl.ds(base, _BPW)])

  return k(bp, g, gw_pad)


def _tc_body(xs_ref, win_ref, wg_ref, g_ref, y_ref, rw_ref, al_ref, out_ref):
  i = pl.program_id(0)
  xs = xs_ref[...]
  buf = win_ref[...]
  wg = wg_ref[...]
  gcol = g_ref[0]
  ycol = y_ref[0]
  rw = rw_ref[...]
  al = al_ref[...]

  cols = lax.broadcasted_iota(jnp.int32, xs.shape, 1)
  x = jnp.where(cols < _N_GROUPS, xs, -1e30)
  m = jnp.max(x, axis=1, keepdims=True)
  lse = m + jnp.log(jnp.sum(jnp.exp(x - m), axis=1, keepdims=True))
  oh = cols == gcol
  x_y = jnp.sum(jnp.where(oh, x, 0.0), axis=1, keepdims=True)
  w_y = jnp.sum(jnp.where(oh, rw, 0.0), axis=1, keepdims=True)
  a_y = jnp.sum(jnp.where(oh, al, 0.0), axis=1, keepdims=True)
  logp_y = x_y - lse
  smooth_sum = (jnp.sum(xs * rw, axis=1, keepdims=True) - lse * jnp.sum(rw))
  root_loss = (-(1.0 - _ROOT_LS) * logp_y
               - (_ROOT_LS / _N_GROUPS) * smooth_sum / w_y)

  p = (_N_GROUPS + gcol * _LEAVES) & (_W - 1)
  for s in (64, 32, 16, 8, 4):
    rolled = jnp.concatenate([buf[:, s:], buf[:, :s]], axis=1)
    buf = jnp.where((p & s) != 0, rolled, buf)
  glr = buf[:, :_W]

  cols2 = lax.broadcasted_iota(jnp.int32, glr.shape, 1)
  gl = jnp.where(cols2 < _LEAVES, glr, -1e30)
  m2 = jnp.max(gl, axis=1, keepdims=True)
  lse2 = m2 + jnp.log(jnp.sum(jnp.exp(gl - m2), axis=1, keepdims=True))
  oh2 = cols2 == ycol
  gl_y = jnp.sum(jnp.where(oh2, gl, 0.0), axis=1, keepdims=True)
  wg_y = jnp.sum(jnp.where(oh2, wg, 0.0), axis=1, keepdims=True)
  logp_y2 = gl_y - lse2
  smooth_sum2 = (jnp.sum(gl * wg, axis=1, keepdims=True)
                 - lse2 * jnp.sum(wg, axis=1, keepdims=True))
  grp_loss = a_y * (-(1.0 - _GROUP_LS) * logp_y2
                    - (_GROUP_LS / _LEAVES) * smooth_sum2 / wg_y)

  acc = (jnp.sum(root_loss + grp_loss) * (1.0 / _BATCH)).reshape(1, 1)

  @pl.when(i == 0)
  def _():
    out_ref[...] = jnp.zeros_like(out_ref)

  out_ref[...] += acc


def _tc_loss(xs, win, wg, g3, y3, rw2, al2):
  out = pl.pallas_call(
      _tc_body,
      grid=(_NB,),
      in_specs=[
          pl.BlockSpec((_R, _PADC), lambda i: (i, 0)),
          pl.BlockSpec((_R, _WIN), lambda i: (i, 0)),
          pl.BlockSpec((_R, _W), lambda i: (i, 0)),
          pl.BlockSpec((1, _R, 1), lambda i: (i, 0, 0)),
          pl.BlockSpec((1, _R, 1), lambda i: (i, 0, 0)),
          pl.BlockSpec((1, _PADC), lambda i: (0, 0)),
          pl.BlockSpec((1, _PADC), lambda i: (0, 0)),
      ],
      out_specs=pl.BlockSpec((1, 1), lambda i: (0, 0)),
      out_shape=jax.ShapeDtypeStruct((1, 1), jnp.float32),
  )(xs, win, wg, g3, y3, rw2, al2)
  return out[0, 0]


def kernel(batch_predictions, targets, root_weight, group_weights, group_alphas):
  g = (targets // _LEAVES).astype(jnp.int32)
  y = (targets % _LEAVES).astype(jnp.int32)
  gw_pad = jnp.pad(group_weights, ((0, 0), (0, _W - _LEAVES)))
  win, wg = _sc_gather(batch_predictions, g, gw_pad)
  xs = lax.slice(batch_predictions, (0, 0), (_BATCH, _PADC))
  g3 = g.reshape(_NB, _R, 1)
  y3 = y.reshape(_NB, _R, 1)
  rw2 = jnp.pad(root_weight, (0, _PADC - _N_GROUPS)).reshape(1, _PADC)
  al2 = jnp.pad(group_alphas, (0, _PADC - _N_GROUPS)).reshape(1, _PADC)
  return _tc_loss(xs, win, wg, g3, y3, rw2, al2)
